# baseline (speedup 1.0000x reference)
import functools

import jax
import jax.numpy as jnp
from jax import lax
from jax.experimental import pallas as pl
from jax.experimental.pallas import tpu as pltpu
from jax.experimental.pallas import tpu_sc as plsc

EMBED_DIM = 32


@functools.partial(jax.jit, static_argnums=(2, 3, 4, 5))
def _gather_rows(idx, table, B, b_per_w, C, NC):
    mesh = plsc.VectorSubcoreMesh(core_axis_name="c", subcore_axis_name="s")
    n = b_per_w // C
    S = 4
    Cs = C // S

    @functools.partial(
        pl.kernel,
        mesh=mesh,
        out_type=jax.ShapeDtypeStruct((B, EMBED_DIM), jnp.float32),
        scratch_types=[
            pltpu.VMEM((C,), jnp.int32),
            pltpu.VMEM((C,), jnp.int32),
            pltpu.VMEM((C, EMBED_DIM), jnp.float32),
            pltpu.VMEM((C, EMBED_DIM), jnp.float32),
            pltpu.SemaphoreType.DMA,
            pltpu.SemaphoreType.DMA,
            pltpu.SemaphoreType.DMA,
            pltpu.SemaphoreType.DMA,
        ],
        compiler_params=pltpu.CompilerParams(use_tc_tiling_on_sc=False),
    )
    def k(idx_hbm, table_hbm, out_hbm, idx0, idx1, rows0, rows1, g0, g1, s0, s1):
        wid = lax.axis_index("s") * NC + lax.axis_index("c")
        base_w = wid * b_per_w

        def cbase(c):
            return pl.multiple_of(base_w + c * C, 8)

        def gather_start(idx_v, rows_v, g):
            for q in range(S):
                sl = pl.ds(q * Cs, Cs)
                pltpu.async_copy(table_hbm.at[idx_v.at[sl]], rows_v.at[sl], g)

        def gather_wait(idx_v, rows_v, g):
            for q in range(S):
                sl = pl.ds(q * Cs, Cs)
                pltpu.make_async_copy(table_hbm.at[idx_v.at[sl]], rows_v.at[sl], g).wait()

        pltpu.sync_copy(idx_hbm.at[pl.ds(cbase(0), C)], idx0)
        gather_start(idx0, rows0, g0)
        pltpu.sync_copy(idx_hbm.at[pl.ds(cbase(1), C)], idx1)
        gather_start(idx1, rows1, g1)

        def body(j, carry):
            a = 2 * j
            b = a + 1
            gather_wait(idx0, rows0, g0)
            pltpu.async_copy(rows0, out_hbm.at[pl.ds(cbase(a), C)], s0)
            gather_wait(idx1, rows1, g1)
            pltpu.async_copy(rows1, out_hbm.at[pl.ds(cbase(b), C)], s1)

            @pl.when(j < n // 2 - 1)
            def _():
                pltpu.sync_copy(idx_hbm.at[pl.ds(cbase(a + 2), C)], idx0)
                pltpu.make_async_copy(rows0, out_hbm.at[pl.ds(cbase(a), C)], s0).wait()
                gather_start(idx0, rows0, g0)
                pltpu.sync_copy(idx_hbm.at[pl.ds(cbase(b + 2), C)], idx1)
                pltpu.make_async_copy(rows1, out_hbm.at[pl.ds(cbase(b), C)], s1).wait()
                gather_start(idx1, rows1, g1)

            return carry

        lax.fori_loop(0, n // 2, body, 0)
        pltpu.make_async_copy(rows0, out_hbm.at[pl.ds(cbase(n - 2), C)], s0).wait()
        pltpu.make_async_copy(rows1, out_hbm.at[pl.ds(cbase(n - 1), C)], s1).wait()

    return k(idx, table)


def kernel(x, w):
    B = x.shape[0] * x.shape[1]
    idx = x.reshape(B).astype(jnp.int32)
    info = plsc.get_sparse_core_info()
    NC, NS = info.num_cores, info.num_subcores
    b_per_w = B // (NC * NS)
    C = 1600
    out = _gather_rows(idx, w, B, b_per_w, C, NC)
    return out.reshape(x.shape[0], x.shape[1], EMBED_DIM)


# speedup vs baseline: 1.7448x; 1.7448x over previous
import functools

import jax
import jax.numpy as jnp
from jax import lax
from jax.experimental import pallas as pl
from jax.experimental.pallas import tpu as pltpu
from jax.experimental.pallas import tpu_sc as plsc

EMBED_DIM = 32


@functools.partial(jax.jit, static_argnums=(2, 3, 4, 5))
def _gather_rows(idx, table, B, b_per_w, C, NC):
    mesh = plsc.VectorSubcoreMesh(core_axis_name="c", subcore_axis_name="s")
    n = b_per_w // C
    S = 4
    Cs = C // S

    @functools.partial(
        pl.kernel,
        mesh=mesh,
        out_type=jax.ShapeDtypeStruct((B, EMBED_DIM), jnp.float32),
        scratch_types=[
            pltpu.VMEM((C,), jnp.int32),
            pltpu.VMEM((C,), jnp.int32),
            pltpu.VMEM((C, EMBED_DIM), jnp.float32),
            pltpu.VMEM((C, EMBED_DIM), jnp.float32),
            pltpu.SemaphoreType.DMA,
            pltpu.SemaphoreType.DMA,
            pltpu.SemaphoreType.DMA,
            pltpu.SemaphoreType.DMA,
        ],
        compiler_params=pltpu.CompilerParams(use_tc_tiling_on_sc=False),
    )
    def k(idx_hbm, table_hbm, out_hbm, idx0, idx1, rows0, rows1, g0, g1, s0, s1):
        wid = lax.axis_index("s") * NC + lax.axis_index("c")
        base_w = wid * b_per_w

        def cbase(c):
            return pl.multiple_of(base_w + c * C, 8)

        def gather_start(idx_v, rows_v, g):
            for q in range(S):
                sl = pl.ds(q * Cs, Cs)
                pltpu.async_copy(table_hbm.at[idx_v.at[sl]], rows_v.at[sl], g)

        def gather_wait(idx_v, rows_v, g):
            for q in range(S):
                sl = pl.ds(q * Cs, Cs)
                pltpu.make_async_copy(table_hbm.at[idx_v.at[sl]], rows_v.at[sl], g).wait()

        pltpu.sync_copy(idx_hbm.at[pl.ds(cbase(0), C)], idx0)
        gather_start(idx0, rows0, g0)
        pltpu.sync_copy(idx_hbm.at[pl.ds(cbase(1), C)], idx1)
        gather_start(idx1, rows1, g1)

        def body(j, carry):
            a = 2 * j
            b = a + 1
            gather_wait(idx0, rows0, g0)
            pltpu.async_copy(rows0, out_hbm.at[pl.ds(cbase(a), C)], s0)
            gather_wait(idx1, rows1, g1)
            pltpu.async_copy(rows1, out_hbm.at[pl.ds(cbase(b), C)], s1)

            @pl.when(j < n // 2 - 1)
            def _():
                pltpu.sync_copy(idx_hbm.at[pl.ds(cbase(a + 2), C)], idx0)
                pltpu.make_async_copy(rows0, out_hbm.at[pl.ds(cbase(a), C)], s0).wait()
                gather_start(idx0, rows0, g0)
                pltpu.sync_copy(idx_hbm.at[pl.ds(cbase(b + 2), C)], idx1)
                pltpu.make_async_copy(rows1, out_hbm.at[pl.ds(cbase(b), C)], s1).wait()
                gather_start(idx1, rows1, g1)

            return carry

        lax.fori_loop(0, n // 2, body, 0)
        pltpu.make_async_copy(rows0, out_hbm.at[pl.ds(cbase(n - 2), C)], s0).wait()
        pltpu.make_async_copy(rows1, out_hbm.at[pl.ds(cbase(n - 1), C)], s1).wait()

    return k(idx, table)


def kernel(x, w):
    BATCH, HIST = x.shape
    B = BATCH * HIST
    idx = jnp.swapaxes(x, 0, 1).reshape(B).astype(jnp.int32)
    info = plsc.get_sparse_core_info()
    NC, NS = info.num_cores, info.num_subcores
    b_per_w = B // (NC * NS)
    C = 1600
    out = _gather_rows(idx, w, B, b_per_w, C, NC)
    return jnp.swapaxes(out.reshape(HIST, BATCH, EMBED_DIM), 0, 1)
